# baseline (speedup 1.0000x reference)
import functools

import jax
import jax.numpy as jnp
from jax.experimental import pallas as pl
from jax.experimental.pallas import tpu as pltpu

B = 16384
A = 1000
BR = 512


def _dr_block(out_ref, rew_ref, act_ref, delta_ref, prop_ref, acc_ref):
    i = pl.program_id(0)

    o = out_ref[...]
    r = rew_ref[...]
    act = act_ref[0]
    delta = delta_ref[0]
    prop = prop_ref[0]

    m = jnp.max(o, axis=1, keepdims=True)
    e = jnp.exp(o - m)
    s = jnp.sum(e, axis=1)
    dot = jnp.sum(e * r, axis=1)

    col = jax.lax.broadcasted_iota(jnp.int32, (BR, A), 1)
    mask = col == act.reshape(BR, 1)
    ea = jnp.sum(jnp.where(mask, e, 0.0), axis=1)
    ra = jnp.sum(jnp.where(mask, r, 0.0), axis=1)

    d = delta.reshape(BR)
    p = prop.reshape(BR)
    contrib = (dot + ea * (d - ra) / p) / s
    partial = jnp.sum(contrib)

    @pl.when(i == 0)
    def _():
        acc_ref[0, 0] = 0.0

    acc_ref[0, 0] += partial


@jax.jit
def kernel(output, action, delta, prop, reward_estimates):
    g = B // BR
    act3 = action.reshape(g, 1, BR)
    delta3 = delta.reshape(g, 1, BR)
    prop3 = prop.reshape(g, 1, BR)

    row_spec = pl.BlockSpec((BR, A), lambda i: (i, 0))
    vec_spec = pl.BlockSpec((1, 1, BR), lambda i: (i, 0, 0))

    acc = pl.pallas_call(
        _dr_block,
        grid=(g,),
        in_specs=[row_spec, row_spec, vec_spec, vec_spec, vec_spec],
        out_specs=pl.BlockSpec(memory_space=pltpu.SMEM),
        out_shape=jax.ShapeDtypeStruct((1, 1), jnp.float32),
    )(output, reward_estimates, act3, delta3, prop3)

    return -acc[0, 0] / B
